# baseline (speedup 1.0000x reference)
import functools

import jax
import jax.numpy as jnp
from jax import lax
from jax.experimental import pallas as pl
from jax.experimental.pallas import tpu as pltpu
from jax.experimental.pallas import tpu_sc as plsc

N = 10000
E = 320000
H = 128
C = 40
BN_EPS = 1e-5

NC = 2
NS = 16
NW = NC * NS
EPW = E // NW
CHUNK = 128
FULL_CHUNKS = EPW // CHUNK
REM = EPW - FULL_CHUNKS * CHUNK
N_PAD = 10112
ROWS_PER_TILE = N_PAD // NS


_NB = 4


def _agg_body(x_hbm, src_hbm, dst_hbm, zeros_hbm, out_hbm,
              agg_sh, src_q, dst_q, rows_b, srcr_v, dstr_v, rowsr_v,
              sem_i, sem_g, sem_s, sem_r):
    cid = lax.axis_index("c")
    sid = lax.axis_index("s")
    wid = sid * NC + cid
    base = wid * EPW

    pltpu.sync_copy(zeros_hbm.at[pl.ds(sid * ROWS_PER_TILE, ROWS_PER_TILE)],
                    agg_sh.at[pl.ds(sid * ROWS_PER_TILE, ROWS_PER_TILE)])

    def issue_idx(c, q):
        off = base + c * CHUNK
        pltpu.async_copy(src_hbm.at[pl.ds(off, CHUNK)], src_q[q], sem_i[q])
        pltpu.async_copy(dst_hbm.at[pl.ds(off, CHUNK)], dst_q[q], sem_i[q])

    def wait_idx(c, q):
        off = base + c * CHUNK
        pltpu.make_async_copy(src_hbm.at[pl.ds(off, CHUNK)], src_q[q], sem_i[q]).wait()
        pltpu.make_async_copy(dst_hbm.at[pl.ds(off, CHUNK)], dst_q[q], sem_i[q]).wait()

    def start_gather(q):
        pltpu.async_copy(x_hbm.at[src_q[q]], rows_b[q % 2], sem_g[q % 2])

    def wait_gather(q):
        pltpu.make_async_copy(x_hbm.at[src_q[q]], rows_b[q % 2], sem_g[q % 2]).wait()

    def start_scatter(q):
        pltpu.async_copy(rows_b[q % 2], agg_sh.at[dst_q[q]], sem_s[q % 2], add=True)

    def wait_scatter(q):
        pltpu.make_async_copy(rows_b[q % 2], agg_sh.at[dst_q[q]], sem_s[q % 2]).wait()

    def step(c, q, first=False, g_next=True, i_next=True):
        wait_gather(q)
        start_scatter(q)
        if g_next:
            wait_idx(c + 1, (q + 1) % _NB)
        if not first:
            wait_scatter((q + 3) % _NB)
        if g_next:
            start_gather((q + 1) % _NB)
        if i_next:
            issue_idx(c + 2, (q + 2) % _NB)

    issue_idx(0, 0)
    issue_idx(1, 1)
    plsc.subcore_barrier()
    wait_idx(0, 0)
    start_gather(0)
    step(0, 0, first=True)
    step(1, 1)
    step(2, 2)
    step(3, 3)

    def quad_step(i, carry):
        step(4 * i, 0)
        step(4 * i + 1, 1)
        step(4 * i + 2, 2)
        step(4 * i + 3, 3)
        return carry

    lax.fori_loop(1, FULL_CHUNKS // _NB, quad_step, 0)

    step(FULL_CHUNKS - 2, (FULL_CHUNKS - 2) % _NB, i_next=False)
    step(FULL_CHUNKS - 1, (FULL_CHUNKS - 1) % _NB, g_next=False, i_next=False)

    off = base + FULL_CHUNKS * CHUNK
    pltpu.sync_copy(src_hbm.at[pl.ds(off, REM)], srcr_v)
    pltpu.sync_copy(dst_hbm.at[pl.ds(off, REM)], dstr_v)
    pltpu.async_copy(x_hbm.at[srcr_v], rowsr_v, sem_r).wait()
    pltpu.sync_copy(rowsr_v, agg_sh.at[dstr_v], add=True)

    wait_scatter((FULL_CHUNKS - 1) % _NB)

    plsc.subcore_barrier()
    pltpu.sync_copy(agg_sh.at[pl.ds(sid * ROWS_PER_TILE, ROWS_PER_TILE)],
                    out_hbm.at[cid, pl.ds(sid * ROWS_PER_TILE, ROWS_PER_TILE)])


@functools.cache
def _make_agg():
    return pl.kernel(
        _agg_body,
        out_type=jax.ShapeDtypeStruct((NC, N_PAD, H), jnp.float32),
        mesh=plsc.VectorSubcoreMesh(core_axis_name="c", subcore_axis_name="s"),
        scratch_types=[
            pltpu.VMEM_SHARED((N_PAD, H), jnp.float32),
            [pltpu.VMEM((CHUNK,), jnp.int32)] * _NB,
            [pltpu.VMEM((CHUNK,), jnp.int32)] * _NB,
            [pltpu.VMEM((CHUNK, H), jnp.float32)] * 2,
            pltpu.VMEM((REM,), jnp.int32),
            pltpu.VMEM((REM,), jnp.int32),
            pltpu.VMEM((REM, H), jnp.float32),
            [pltpu.SemaphoreType.DMA] * _NB,
            [pltpu.SemaphoreType.DMA] * 2,
            [pltpu.SemaphoreType.DMA] * 2,
            pltpu.SemaphoreType.DMA,
        ],
    )


def _agg(x, src, dst, zeros):
    p = _make_agg()(x, src, dst, zeros)
    return p[:, :N]


def _mlp_bn_body(eps_ref, x_ref, p0_ref, p1_ref, Wa_ref, ba_ref, Wb_ref,
                 bb_ref, g_ref, beta_ref, out_ref):
    h = x_ref[...] * (1.0 + eps_ref[0]) + (p0_ref[...] + p1_ref[...])
    h = jnp.maximum(jnp.dot(h, Wa_ref[...], preferred_element_type=jnp.float32)
                    + ba_ref[...], 0.0)
    h = jnp.maximum(jnp.dot(h, Wb_ref[...], preferred_element_type=jnp.float32)
                    + bb_ref[...], 0.0)
    mean = jnp.mean(h, axis=0, keepdims=True)
    var = jnp.mean(jnp.square(h - mean), axis=0, keepdims=True)
    out_ref[...] = (h - mean) * lax.rsqrt(var + BN_EPS) * g_ref[...] + beta_ref[...]


def _head_body(eps_ref, x_ref, p0_ref, p1_ref, Wa_ref, ba_ref, Wb_ref,
               bb_ref, g_ref, beta_ref, Wl1_ref, bl1_ref, Wl2_ref, bl2_ref,
               out_ref):
    h = x_ref[...] * (1.0 + eps_ref[0]) + (p0_ref[...] + p1_ref[...])
    h = jnp.maximum(jnp.dot(h, Wa_ref[...], preferred_element_type=jnp.float32)
                    + ba_ref[...], 0.0)
    h = jnp.maximum(jnp.dot(h, Wb_ref[...], preferred_element_type=jnp.float32)
                    + bb_ref[...], 0.0)
    mean = jnp.mean(h, axis=0, keepdims=True)
    var = jnp.mean(jnp.square(h - mean), axis=0, keepdims=True)
    h = (h - mean) * lax.rsqrt(var + BN_EPS) * g_ref[...] + beta_ref[...]
    h = jnp.maximum(jnp.dot(h, Wl1_ref[...], preferred_element_type=jnp.float32)
                    + bl1_ref[...], 0.0)
    out_ref[...] = (jnp.dot(h, Wl2_ref[...], preferred_element_type=jnp.float32)
                    + bl2_ref[...])


_SMEM1 = pl.BlockSpec(memory_space=pltpu.SMEM)


def _mlp_bn(eps, x, p0, p1, Wa, ba, Wb, bb, g, beta):
    return pl.pallas_call(
        _mlp_bn_body,
        out_shape=jax.ShapeDtypeStruct((N, H), jnp.float32),
        in_specs=[_SMEM1] + [pl.BlockSpec()] * 9,
        out_specs=pl.BlockSpec(),
    )(eps.reshape(1), x, p0, p1, Wa, ba, Wb, bb, g, beta)


def _head(eps, x, p0, p1, Wa, ba, Wb, bb, g, beta, Wl1, bl1, Wl2, bl2):
    return pl.pallas_call(
        _head_body,
        out_shape=jax.ShapeDtypeStruct((N, C), jnp.float32),
        in_specs=[_SMEM1] + [pl.BlockSpec()] * 13,
        out_specs=pl.BlockSpec(),
    )(eps.reshape(1), x, p0, p1, Wa, ba, Wb, bb, g, beta, Wl1, bl1, Wl2, bl2)


def kernel(x, edge_index, eps0, W0a, b0a, W0b, b0b, g0, beta0,
           eps1, W1a, b1a, W1b, b1b, g1, beta1,
           eps2, W2a, b2a, W2b, b2b, g2, beta2, Wl1, bl1, Wl2, bl2):
    src = edge_index[0]
    dst = edge_index[1]
    zeros = jnp.zeros((N_PAD, H), jnp.float32)

    p = _agg(x, src, dst, zeros)
    h = _mlp_bn(eps0, x, p[0], p[1], W0a, b0a, W0b, b0b, g0, beta0)
    p = _agg(h, src, dst, zeros)
    h = _mlp_bn(eps1, h, p[0], p[1], W1a, b1a, W1b, b1b, g1, beta1)
    p = _agg(h, src, dst, zeros)
    return _head(eps2, h, p[0], p[1], W2a, b2a, W2b, b2b, g2, beta2,
                 Wl1, bl1, Wl2, bl2)
